# Initial kernel scaffold; baseline (speedup 1.0000x reference)
import jax
import jax.numpy as jnp
from jax.experimental import pallas as pl


def kernel(inputs, Wc, bc, W_in, b_in, W_blk, b_blk, W_out, b_out):
    raise NotImplementedError("write your pallas kernel here")


# speedup vs baseline: 5.3822x; 5.3822x over previous
import functools

import jax
import jax.numpy as jnp
from jax import lax
from jax.experimental import pallas as pl
from jax.experimental.pallas import tpu as pltpu
from jax.experimental.pallas import tpu_sc as plsc

_CAPS = (4096, 4096, 4096, 4096)

_P = 64
_NC = 2
_NS = 16
_NW = _NC * _NS


def _cdiv(a, b):
    return (a + b - 1) // b


def _classifier_call(inputs, Wc, bc, cblk):
    n, c, h, w = inputs.shape
    e = Wc.shape[0]
    g = n // cblk

    def body(x_ref, wc_ref, bc_ref, route_ref, xt_ref):
        x = x_ref[...]
        pooled = jnp.mean(x, axis=(2, 3))
        scores = jnp.dot(pooled, wc_ref[...].T,
                         preferred_element_type=jnp.float32) + bc_ref[...][None, :]
        route_ref[0, 0, :] = jnp.argmax(scores, axis=-1).astype(jnp.int32)
        xt_ref[...] = x.transpose(0, 2, 1, 3).reshape(cblk, h, c * w)

    route3, xt = pl.pallas_call(
        body,
        grid=(g,),
        in_specs=[
            pl.BlockSpec((cblk, c, h, w), lambda i: (i, 0, 0, 0)),
            pl.BlockSpec((e, c), lambda i: (0, 0)),
            pl.BlockSpec((e,), lambda i: (0,)),
        ],
        out_specs=[
            pl.BlockSpec((1, 1, cblk), lambda i: (i, 0, 0)),
            pl.BlockSpec((cblk, h, c * w), lambda i: (i, 0, 0)),
        ],
        out_shape=[
            jax.ShapeDtypeStruct((g, 1, cblk), jnp.int32),
            jax.ShapeDtypeStruct((n, h, c * w), jnp.float32),
        ],
        compiler_params=pltpu.CompilerParams(
            dimension_semantics=("arbitrary",),
            vmem_limit_bytes=100 * 1024 * 1024),
    )(inputs, Wc, bc)
    return route3.reshape(n), xt


def _routing_metadata(route, n, e, p, g_static, npad, caps):
    order = jnp.argsort(route, stable=True).astype(jnp.int32)
    route_sorted = route[order]
    counts = jnp.sum(route[:, None] == jnp.arange(e)[None, :], axis=0)
    offsets = jnp.concatenate([jnp.zeros((1,), jnp.int32),
                               jnp.cumsum(counts)[:-1].astype(jnp.int32)])
    nb = _cdiv(counts, p)
    bstart = jnp.concatenate([jnp.zeros((1,), jnp.int32),
                              jnp.cumsum(nb)[:-1].astype(jnp.int32)])
    i = jnp.arange(n, dtype=jnp.int32)
    rank = i - offsets[route_sorted]
    slots = bstart[route_sorted] * p + rank
    caps_arr = jnp.asarray(caps, jnp.int32)
    src_idx = jnp.zeros((npad,), jnp.int32).at[slots].set(order)
    dst_idx = jnp.full((npad,), n, jnp.int32).at[slots].set(order)
    valid = jnp.zeros((npad,), jnp.float32).at[slots].set(
        (rank < caps_arr[route_sorted]).astype(jnp.float32))
    gidx = jnp.arange(g_static, dtype=jnp.int32)
    eob = jnp.clip(
        jnp.sum(gidx[:, None] >= bstart[None, :], axis=1) - 1, 0, e - 1
    ).astype(jnp.int32)
    return src_idx, dst_idx, valid, eob


def _tridiag_weights(w_conv, hw):
    ar = jnp.arange(hw)
    s = jnp.stack([(ar[:, None] - ar[None, :] == dw - 1) for dw in range(3)])
    s = s.astype(jnp.float32)
    lead = w_conv.shape[:-4]
    w2 = w_conv.reshape((-1,) + w_conv.shape[-4:])
    wt = jnp.einsum('aoixy,ywv->axiwov', w2, s)
    co, ci = w_conv.shape[-4], w_conv.shape[-3]
    return wt.reshape(lead + (3, ci * hw, co * hw))


def _sc_gather(table, idx3, npad, d, nchunks, csz):
    mesh = plsc.VectorSubcoreMesh(core_axis_name="c", subcore_axis_name="s")

    @functools.partial(
        pl.kernel, mesh=mesh,
        out_type=jax.ShapeDtypeStruct((npad, d), jnp.float32),
        scratch_types=[
            pltpu.VMEM((nchunks, csz), jnp.int32),
            pltpu.VMEM((csz, d), jnp.float32),
            pltpu.SemaphoreType.DMA,
        ],
    )
    def k(table_hbm, idx_hbm, out_hbm, idx_v, rows_v, sem):
        wid = lax.axis_index("s") * _NC + lax.axis_index("c")
        base = wid * (nchunks * csz)
        pltpu.sync_copy(idx_hbm.at[wid], idx_v)
        for j in range(nchunks):
            pltpu.async_copy(table_hbm.at[idx_v.at[j]], rows_v, sem).wait()
            pltpu.sync_copy(rows_v, out_hbm.at[pl.ds(base + j * csz, csz)])

    return k(table, idx3)


def _sc_scatter(rows, idx3, nout, d, nchunks, csz):
    mesh = plsc.VectorSubcoreMesh(core_axis_name="c", subcore_axis_name="s")

    @functools.partial(
        pl.kernel, mesh=mesh,
        out_type=jax.ShapeDtypeStruct((nout, d), jnp.float32),
        scratch_types=[
            pltpu.VMEM((nchunks, csz), jnp.int32),
            pltpu.VMEM((csz, d), jnp.float32),
            pltpu.SemaphoreType.DMA,
        ],
    )
    def k(rows_hbm, idx_hbm, out_hbm, idx_v, rows_v, sem):
        wid = lax.axis_index("s") * _NC + lax.axis_index("c")
        base = wid * (nchunks * csz)
        pltpu.sync_copy(idx_hbm.at[wid], idx_v)
        for j in range(nchunks):
            pltpu.sync_copy(rows_hbm.at[pl.ds(base + j * csz, csz)], rows_v)
            pltpu.async_copy(rows_v, out_hbm.at[idx_v.at[j]], sem).wait()

    return k(rows, idx3)


def _expert_call(x_perm, eob, valid, wt_h, wt_b, wt_t, bh, bb, bt,
                 p, g, hw, cin, f, nblk):
    kin = cin * hw
    kmid = f * hw

    def conv3(x, wt3, brow, kout):
        zero = jnp.zeros((p, 1, x.shape[-1]), jnp.float32)
        xp = jnp.concatenate([zero, x, zero], axis=1)
        acc = None
        for dh in range(3):
            a = xp[:, dh:dh + hw, :].reshape(p * hw, x.shape[-1])
            t = jnp.dot(a, wt3[dh], preferred_element_type=jnp.float32)
            acc = t if acc is None else acc + t
        return acc.reshape(p, hw, kout) + brow[None, None, :]

    def body(eob_ref, x_ref, wh_ref, wb_ref, wt_ref, bh_ref, bb_ref, bt_ref,
             v_ref, y_ref):
        x = x_ref[...]
        h = jax.nn.relu(conv3(x, wh_ref[0], bh_ref[0, 0], kmid))
        for i in range(nblk):
            h = h + jax.nn.relu(conv3(h, wb_ref[0, i], bb_ref[0, i, 0], kmid))
        y = conv3(h, wt_ref[0], bt_ref[0, 0], kin)
        y_ref[...] = y * v_ref[0, 0, :][:, None, None]

    grid_spec = pltpu.PrefetchScalarGridSpec(
        num_scalar_prefetch=1,
        grid=(g,),
        in_specs=[
            pl.BlockSpec((p, hw, kin), lambda i, e_r: (i, 0, 0)),
            pl.BlockSpec((1, 3, kin, kmid), lambda i, e_r: (e_r[i], 0, 0, 0)),
            pl.BlockSpec((1, nblk, 3, kmid, kmid),
                         lambda i, e_r: (e_r[i], 0, 0, 0, 0)),
            pl.BlockSpec((1, 3, kmid, kin), lambda i, e_r: (e_r[i], 0, 0, 0)),
            pl.BlockSpec((1, 1, kmid), lambda i, e_r: (e_r[i], 0, 0)),
            pl.BlockSpec((1, nblk, 1, kmid), lambda i, e_r: (e_r[i], 0, 0, 0)),
            pl.BlockSpec((1, 1, kin), lambda i, e_r: (e_r[i], 0, 0)),
            pl.BlockSpec((1, 1, p), lambda i, e_r: (i, 0, 0)),
        ],
        out_specs=pl.BlockSpec((p, hw, kin), lambda i, e_r: (i, 0, 0)),
    )
    return pl.pallas_call(
        body,
        grid_spec=grid_spec,
        out_shape=jax.ShapeDtypeStruct((g * p, hw, kin), jnp.float32),
        compiler_params=pltpu.CompilerParams(
            dimension_semantics=("arbitrary",),
            vmem_limit_bytes=100 * 1024 * 1024),
    )(eob, x_perm, wt_h, wt_b, wt_t, bh, bb, bt, valid)


def kernel(inputs, Wc, bc, W_in, b_in, W_blk, b_blk, W_out, b_out):
    n, cin, hw, _ = inputs.shape
    e = Wc.shape[0]
    f = W_in.shape[1]
    nblk = W_blk.shape[1]
    d = cin * hw * hw

    p = _P
    g = n // p + e
    npad = g * p

    rpw = npad // _NW
    csz = 24
    while rpw % csz:
        csz -= 8
    nchunks = rpw // csz

    route, inputs_t = _classifier_call(inputs, Wc, bc, cblk=256)

    src_idx, dst_idx, valid, eob = _routing_metadata(
        route, n, e, p, g, npad, _CAPS)
    src3 = src_idx.reshape(_NW, nchunks, csz)
    dst3 = dst_idx.reshape(_NW, nchunks, csz)
    valid3 = valid.reshape(g, 1, p)

    wt_h = _tridiag_weights(W_in, hw)
    wt_b = _tridiag_weights(W_blk, hw)
    wt_t = _tridiag_weights(W_out, hw)
    bh = jnp.repeat(b_in, hw, axis=1).reshape(e, 1, f * hw)
    bb = jnp.repeat(b_blk, hw, axis=2).reshape(e, nblk, 1, f * hw)
    bt = jnp.repeat(b_out, hw, axis=1).reshape(e, 1, cin * hw)

    x_perm = _sc_gather(inputs_t.reshape(n, d), src3, npad, d, nchunks, csz)

    y_perm = _expert_call(x_perm.reshape(npad, hw, cin * hw), eob, valid3,
                          wt_h, wt_b, wt_t, bh, bb, bt, p, g, hw, cin, f, nblk)

    out_t = _sc_scatter(y_perm.reshape(npad, d), dst3, n + 8, d, nchunks, csz)

    return out_t[:n].reshape(n, hw, cin, hw).transpose(0, 2, 1, 3)
